# baseline (speedup 1.0000x reference)
import functools
import jax
import jax.numpy as jnp
from jax import lax
from jax.experimental import pallas as pl
from jax.experimental.pallas import tpu as pltpu
from jax.experimental.pallas import tpu_sc as plsc

_NC = 2
_NS = 16
_R = 400


def _prep1_body(x_ref, w1_ref, ssrc_ref, sdst_ref,
                h_ref, as_ref, ad_ref, shift_ref, msrc, mdst):
    i = pl.program_id(0)
    h = jnp.dot(x_ref[...], w1_ref[...], preferred_element_type=jnp.float32)
    h_ref[...] = h
    a_s = jnp.dot(h, ssrc_ref[...], preferred_element_type=jnp.float32,
                  precision=lax.Precision.HIGHEST)
    a_d = jnp.dot(h, sdst_ref[...], preferred_element_type=jnp.float32,
                  precision=lax.Precision.HIGHEST)
    as_ref[...] = a_s
    ad_ref[...] = a_d
    bs = jnp.max(a_s, axis=0, keepdims=True)
    bd = jnp.max(a_d, axis=0, keepdims=True)

    @pl.when(i == 0)
    def _():
        msrc[...] = bs
        mdst[...] = bd

    @pl.when(i > 0)
    def _():
        msrc[...] = jnp.maximum(msrc[...], bs)
        mdst[...] = jnp.maximum(mdst[...], bd)

    @pl.when(i == pl.num_programs(0) - 1)
    def _():
        shift_ref[...] = msrc[...] + mdst[...]


def _prep2_body(part_ref, b1_ref, w2_ref, r8_ref, ms_ref, md_ref,
                h2_ref, as_ref, ad_ref, shift_ref, msrc, mdst):
    i = pl.program_id(0)
    acc = part_ref[0] + part_ref[1]
    num = acc[:, :128]
    den = jnp.dot(acc[:, 128:136], r8_ref[...],
                  preferred_element_type=jnp.float32)
    hmid = jnp.maximum(num / (den + 1e-16) + b1_ref[...], 0.0)
    h2 = jnp.dot(hmid, w2_ref[...], preferred_element_type=jnp.float32)
    h2_ref[...] = h2
    a_s = jnp.dot(h2, ms_ref[...], preferred_element_type=jnp.float32,
                  precision=lax.Precision.HIGHEST)
    a_d = jnp.dot(h2, md_ref[...], preferred_element_type=jnp.float32,
                  precision=lax.Precision.HIGHEST)
    as_ref[...] = a_s
    ad_ref[...] = a_d
    bs = jnp.max(a_s, axis=0, keepdims=True)
    bd = jnp.max(a_d, axis=0, keepdims=True)

    @pl.when(i == 0)
    def _():
        msrc[...] = bs
        mdst[...] = bd

    @pl.when(i > 0)
    def _():
        msrc[...] = jnp.maximum(msrc[...], bs)
        mdst[...] = jnp.maximum(mdst[...], bd)

    @pl.when(i == pl.num_programs(0) - 1)
    def _():
        shift_ref[...] = msrc[...] + mdst[...]


def _final_body(part_ref, b2_ref, out_ref):
    acc = part_ref[0] + part_ref[1]
    out_ref[...] = acc[:, :16] / (acc[:, 16:17] + 1e-16) + b2_ref[...]


def _make_edge_pass(n, e, hc, sc_k):
    rw = hc + 16
    nw = _NC * _NS
    ep = e // nw
    qb = 80
    b = qb * sc_k
    rnds = ep // b
    assert rnds * b == ep and rnds % 2 == 1
    n_pad = ((n + _NS * qb - 1) // (_NS * qb)) * (_NS * qb)
    rpt = n_pad // _NS
    nh = hc // 16

    mesh = plsc.VectorSubcoreMesh(core_axis_name="c", subcore_axis_name="s")

    @functools.partial(
        pl.kernel,
        out_type=jax.ShapeDtypeStruct((_NC, n_pad, rw), jnp.float32),
        mesh=mesh,
        compiler_params=pltpu.CompilerParams(
            use_tc_tiling_on_sc=False, needs_layout_passes=False),
        scratch_types=[
            pltpu.VMEM_SHARED((n_pad, rw), jnp.float32),
            pltpu.VMEM((b,), jnp.int32),
            pltpu.VMEM((b,), jnp.int32),
            pltpu.VMEM((sc_k, qb), jnp.int32),
            pltpu.VMEM((sc_k, qb), jnp.int32),
            pltpu.VMEM((b, 16), jnp.float32),
            pltpu.VMEM((b, 16), jnp.float32),
            pltpu.VMEM((b, 16), jnp.float32),
            pltpu.VMEM((b, 16), jnp.float32),
            pltpu.VMEM((b, hc), jnp.float32),
            pltpu.VMEM((b, hc), jnp.float32),
            pltpu.VMEM((b, rw), jnp.float32),
            pltpu.VMEM((16,), jnp.float32),
            pltpu.SemaphoreType.DMA,
            pltpu.SemaphoreType.DMA,
            pltpu.SemaphoreType.DMA,
            pltpu.SemaphoreType.DMA,
        ],
    )
    def edge_pass(ei, a_src, a_dst, h, shift, part, acc,
                  s0, s1, d0, d1, as0, as1, ad0, ad1, hg0, hg1,
                  ob, shb, si0, si1, sg0, sg1):
        sbuf, dbuf = (s0, s1), (d0, d1)
        asg, adg, hgb = (as0, as1), (ad0, ad1), (hg0, hg1)
        si, sg = (si0, si1), (sg0, sg1)
        c = lax.axis_index("c")
        s = lax.axis_index("s")
        wid = c * _NS + s
        row0 = s * rpt

        zero = jnp.zeros((16,), jnp.float32)

        def zrow(r, _):
            for k in range(rw // 16):
                ob[r, pl.ds(k * 16, 16)] = zero
            return 0

        lax.fori_loop(0, qb, zrow, 0)
        zb = ob.at[pl.ds(0, qb)]
        for z in range(rpt // qb):
            pltpu.sync_copy(zb, acc.at[pl.ds(row0 + z * qb, qb)])
        plsc.subcore_barrier()

        pltpu.sync_copy(shift, shb)

        ebase = wid * ep

        def issue_idx(r, par):
            b0 = pl.multiple_of(ebase + r * b, 8)
            pltpu.async_copy(ei.at[0, pl.ds(b0, b)], sbuf[par], si[par])
            for j in range(sc_k):
                pltpu.async_copy(ei.at[1, pl.ds(b0 + j * qb, qb)],
                                 dbuf[par].at[j], si[par])

        def wait_idx(par):
            pltpu.make_async_copy(ei.at[0, pl.ds(0, b)], sbuf[par],
                                  si[par]).wait()
            for j in range(sc_k):
                pltpu.make_async_copy(ei.at[1, pl.ds(0, qb)],
                                      dbuf[par].at[j], si[par]).wait()

        def issue_gathers(par):
            for j in range(sc_k):
                sl = pl.ds(j * qb, qb)
                pltpu.async_copy(a_src.at[sbuf[par].at[sl]],
                                 asg[par].at[sl], sg[par])
                pltpu.async_copy(a_dst.at[dbuf[par].at[j]],
                                 adg[par].at[sl], sg[par])
                pltpu.async_copy(h.at[sbuf[par].at[sl]],
                                 hgb[par].at[sl], sg[par])

        def wait_gathers(par):
            for j in range(sc_k):
                sl = pl.ds(j * qb, qb)
                pltpu.make_async_copy(a_src.at[sbuf[par].at[sl]],
                                      asg[par].at[sl], sg[par]).wait()
                pltpu.make_async_copy(a_dst.at[dbuf[par].at[j]],
                                      adg[par].at[sl], sg[par]).wait()
                pltpu.make_async_copy(h.at[sbuf[par].at[sl]],
                                      hgb[par].at[sl], sg[par]).wait()

        def compute_scatter(par):
            sh = shb[...]
            hgp = hgb[par]

            @plsc.parallel_loop(0, b, unroll=16)
            def edge(i):
                ev = asg[par][i, :] + adg[par][i, :]
                ev = jnp.maximum(ev, 0.2 * ev)
                p = jnp.exp(ev - sh)
                ob[i, pl.ds(hc, 16)] = p
                for k in range(nh):
                    ph = lax.gather(
                        p, jnp.full((16, 1), k, jnp.int32),
                        dimension_numbers=lax.GatherDimensionNumbers(
                            offset_dims=(), collapsed_slice_dims=(0,),
                            start_index_map=(0,)),
                        slice_sizes=(1,),
                        mode=lax.GatherScatterMode.PROMISE_IN_BOUNDS)
                    ob[i, pl.ds(k * 16, 16)] = ph * hgp[i, pl.ds(k * 16, 16)]

            for j in range(sc_k):
                pltpu.sync_copy(ob.at[pl.ds(j * qb, qb)],
                                acc.at[dbuf[par].at[j]], add=True)

        pltpu.sync_copy(ei.at[0, pl.ds(ebase, b)], sbuf[0])
        for j in range(sc_k):
            pltpu.sync_copy(ei.at[1, pl.ds(ebase + j * qb, qb)],
                            dbuf[0].at[j])
        issue_gathers(0)
        issue_idx(1, 1)

        def sub_body(r, par):
            wait_idx(1 - par)
            issue_gathers(1 - par)
            wait_gathers(par)
            compute_scatter(par)
            issue_idx(lax.rem(r + 2, rnds), par)

        def pair_body(rr, _):
            sub_body(2 * rr, 0)
            sub_body(2 * rr + 1, 1)
            return 0

        lax.fori_loop(0, (rnds - 1) // 2, pair_body, 0)
        wait_gathers(0)
        compute_scatter(0)
        wait_idx(1)
        plsc.subcore_barrier()

        for z in range(rpt // qb):
            r0 = row0 + z * qb
            pltpu.sync_copy(acc.at[pl.ds(r0, qb)], part.at[c, pl.ds(r0, qb)])

    return edge_pass


def kernel(x, edge_index, W1, att_src1, att_dst1, b1,
           W2, att_src2, att_dst2, b2):
    n, _ = x.shape
    e = edge_index.shape[1]
    heads, ch = att_src1.shape
    hc1 = heads * ch
    grid = (n // _R,)

    ar = jnp.arange(hc1)
    ssrc1 = jnp.zeros((hc1, 16), jnp.float32).at[ar, ar // ch].set(
        att_src1.reshape(-1))
    sdst1 = jnp.zeros((hc1, 16), jnp.float32).at[ar, ar // ch].set(
        att_dst1.reshape(-1))
    r8 = jnp.zeros((heads, hc1), jnp.float32).at[ar // ch, ar].set(1.0)
    ms2 = jnp.zeros((ch, 16), jnp.float32).at[jnp.arange(ch), 0].set(
        att_src2[0])
    md2 = jnp.zeros((ch, 16), jnp.float32).at[jnp.arange(ch), 0].set(
        att_dst2[0])

    f32 = jnp.float32
    h1, as1, ad1, shift1 = pl.pallas_call(
        _prep1_body,
        grid=grid,
        in_specs=[
            pl.BlockSpec((_R, 128), lambda i: (i, 0)),
            pl.BlockSpec((128, 128), lambda i: (0, 0)),
            pl.BlockSpec((128, 16), lambda i: (0, 0)),
            pl.BlockSpec((128, 16), lambda i: (0, 0)),
        ],
        out_specs=[
            pl.BlockSpec((_R, 128), lambda i: (i, 0)),
            pl.BlockSpec((_R, 16), lambda i: (i, 0)),
            pl.BlockSpec((_R, 16), lambda i: (i, 0)),
            pl.BlockSpec((1, 16), lambda i: (0, 0)),
        ],
        out_shape=[
            jax.ShapeDtypeStruct((n, 128), f32),
            jax.ShapeDtypeStruct((n, 16), f32),
            jax.ShapeDtypeStruct((n, 16), f32),
            jax.ShapeDtypeStruct((1, 16), f32),
        ],
        scratch_shapes=[pltpu.VMEM((1, 16), f32), pltpu.VMEM((1, 16), f32)],
    )(x, W1, ssrc1, sdst1)

    part1 = _make_edge_pass(n, e, hc1, 1)(
        edge_index, as1, ad1, h1, shift1.reshape(-1))

    h2, as2, ad2, shift2 = pl.pallas_call(
        _prep2_body,
        grid=grid,
        in_specs=[
            pl.BlockSpec((2, _R, 144), lambda i: (0, i, 0)),
            pl.BlockSpec((1, 128), lambda i: (0, 0)),
            pl.BlockSpec((128, 16), lambda i: (0, 0)),
            pl.BlockSpec((8, 128), lambda i: (0, 0)),
            pl.BlockSpec((16, 16), lambda i: (0, 0)),
            pl.BlockSpec((16, 16), lambda i: (0, 0)),
        ],
        out_specs=[
            pl.BlockSpec((_R, 16), lambda i: (i, 0)),
            pl.BlockSpec((_R, 16), lambda i: (i, 0)),
            pl.BlockSpec((_R, 16), lambda i: (i, 0)),
            pl.BlockSpec((1, 16), lambda i: (0, 0)),
        ],
        out_shape=[
            jax.ShapeDtypeStruct((n, 16), f32),
            jax.ShapeDtypeStruct((n, 16), f32),
            jax.ShapeDtypeStruct((n, 16), f32),
            jax.ShapeDtypeStruct((1, 16), f32),
        ],
        scratch_shapes=[pltpu.VMEM((1, 16), f32), pltpu.VMEM((1, 16), f32)],
    )(part1, b1.reshape(1, -1), W2, r8, ms2, md2)

    part2 = _make_edge_pass(n, e, ch, 5)(
        edge_index, as2, ad2, h2, shift2.reshape(-1))

    out = pl.pallas_call(
        _final_body,
        grid=grid,
        in_specs=[
            pl.BlockSpec((2, _R, 32), lambda i: (0, i, 0)),
            pl.BlockSpec((1, 16), lambda i: (0, 0)),
        ],
        out_specs=pl.BlockSpec((_R, 16), lambda i: (i, 0)),
        out_shape=jax.ShapeDtypeStruct((n, 16), f32),
    )(part2, b2.reshape(1, -1))
    return out


# speedup vs baseline: 1.1328x; 1.1328x over previous
import functools
import jax
import jax.numpy as jnp
from jax import lax
from jax.experimental import pallas as pl
from jax.experimental.pallas import tpu as pltpu
from jax.experimental.pallas import tpu_sc as plsc

_NC = 2
_NS = 16
_R = 400


def _prep1_body(x_ref, w1_ref, ssrc_ref, sdst_ref,
                h_ref, as_ref, ad_ref, shift_ref, msrc, mdst):
    i = pl.program_id(0)
    h = jnp.dot(x_ref[...], w1_ref[...], preferred_element_type=jnp.float32)
    h_ref[...] = h
    a_s = jnp.dot(h, ssrc_ref[...], preferred_element_type=jnp.float32,
                  precision=lax.Precision.HIGHEST)
    a_d = jnp.dot(h, sdst_ref[...], preferred_element_type=jnp.float32,
                  precision=lax.Precision.HIGHEST)
    as_ref[...] = a_s
    ad_ref[...] = a_d
    bs = jnp.max(a_s, axis=0, keepdims=True)
    bd = jnp.max(a_d, axis=0, keepdims=True)

    @pl.when(i == 0)
    def _():
        msrc[...] = bs
        mdst[...] = bd

    @pl.when(i > 0)
    def _():
        msrc[...] = jnp.maximum(msrc[...], bs)
        mdst[...] = jnp.maximum(mdst[...], bd)

    @pl.when(i == pl.num_programs(0) - 1)
    def _():
        shift_ref[...] = msrc[...] + mdst[...]


def _prep2_body(part_ref, b1_ref, w2_ref, r8_ref, ms_ref, md_ref,
                h2_ref, as_ref, ad_ref, shift_ref, msrc, mdst):
    i = pl.program_id(0)
    acc = part_ref[0] + part_ref[1]
    num = acc[:, :128]
    den = jnp.dot(acc[:, 128:136], r8_ref[...],
                  preferred_element_type=jnp.float32)
    hmid = jnp.maximum(num / (den + 1e-16) + b1_ref[...], 0.0)
    h2 = jnp.dot(hmid, w2_ref[...], preferred_element_type=jnp.float32)
    h2_ref[...] = h2
    a_s = jnp.dot(h2, ms_ref[...], preferred_element_type=jnp.float32,
                  precision=lax.Precision.HIGHEST)
    a_d = jnp.dot(h2, md_ref[...], preferred_element_type=jnp.float32,
                  precision=lax.Precision.HIGHEST)
    as_ref[...] = a_s
    ad_ref[...] = a_d
    bs = jnp.max(a_s, axis=0, keepdims=True)
    bd = jnp.max(a_d, axis=0, keepdims=True)

    @pl.when(i == 0)
    def _():
        msrc[...] = bs
        mdst[...] = bd

    @pl.when(i > 0)
    def _():
        msrc[...] = jnp.maximum(msrc[...], bs)
        mdst[...] = jnp.maximum(mdst[...], bd)

    @pl.when(i == pl.num_programs(0) - 1)
    def _():
        shift_ref[...] = msrc[...] + mdst[...]


def _final_body(part_ref, b2_ref, out_ref):
    acc = part_ref[0] + part_ref[1]
    out_ref[...] = acc[:, :16] / (acc[:, 16:17] + 1e-16) + b2_ref[...]


def _make_edge_pass(n, e, hc, sc_k):
    rw = hc + 16
    nw = _NC * _NS
    ep = e // nw
    qb = 80
    b = qb * sc_k
    rnds = ep // b
    assert rnds * b == ep and rnds % 2 == 1
    n_pad = ((n + _NS * qb - 1) // (_NS * qb)) * (_NS * qb)
    rpt = n_pad // _NS
    nh = hc // 16

    mesh = plsc.VectorSubcoreMesh(core_axis_name="c", subcore_axis_name="s")

    @functools.partial(
        pl.kernel,
        out_type=jax.ShapeDtypeStruct((_NC, n_pad, rw), jnp.float32),
        mesh=mesh,
        compiler_params=pltpu.CompilerParams(
            use_tc_tiling_on_sc=False, needs_layout_passes=False),
        scratch_types=[
            pltpu.VMEM_SHARED((n_pad, rw), jnp.float32),
            pltpu.VMEM((b,), jnp.int32),
            pltpu.VMEM((b,), jnp.int32),
            pltpu.VMEM((sc_k, qb), jnp.int32),
            pltpu.VMEM((sc_k, qb), jnp.int32),
            pltpu.VMEM((b, 16), jnp.float32),
            pltpu.VMEM((b, 16), jnp.float32),
            pltpu.VMEM((b, 16), jnp.float32),
            pltpu.VMEM((b, 16), jnp.float32),
            pltpu.VMEM((b, hc), jnp.float32),
            pltpu.VMEM((b, hc), jnp.float32),
            pltpu.VMEM((b, rw), jnp.float32),
            pltpu.VMEM((16,), jnp.float32),
            pltpu.SemaphoreType.DMA,
            pltpu.SemaphoreType.DMA,
            pltpu.SemaphoreType.DMA,
            pltpu.SemaphoreType.DMA,
            pltpu.SemaphoreType.DMA,
        ],
    )
    def edge_pass(ei, a_src, a_dst, h, shift, part, acc,
                  s0, s1, d0, d1, as0, as1, ad0, ad1, hg0, hg1,
                  ob, shb, si0, si1, sg0, sg1, ss):
        sbuf, dbuf = (s0, s1), (d0, d1)
        asg, adg, hgb = (as0, as1), (ad0, ad1), (hg0, hg1)
        si, sg = (si0, si1), (sg0, sg1)
        c = lax.axis_index("c")
        s = lax.axis_index("s")
        wid = c * _NS + s
        row0 = s * rpt

        zero = jnp.zeros((16,), jnp.float32)

        def zrow(r, _):
            for k in range(rw // 16):
                ob[r, pl.ds(k * 16, 16)] = zero
            return 0

        lax.fori_loop(0, b, zrow, 0)
        zb = ob.at[pl.ds(0, qb)]
        for z in range(rpt // qb):
            pltpu.sync_copy(zb, acc.at[pl.ds(row0 + z * qb, qb)])
        plsc.subcore_barrier()

        pltpu.sync_copy(shift, shb)

        ebase = wid * ep

        def issue_idx(r, par):
            b0 = pl.multiple_of(ebase + r * b, 8)
            pltpu.async_copy(ei.at[0, pl.ds(b0, b)], sbuf[par], si[par])
            for j in range(sc_k):
                pltpu.async_copy(ei.at[1, pl.ds(b0 + j * qb, qb)],
                                 dbuf[par].at[j], si[par])

        def wait_idx(par):
            pltpu.make_async_copy(ei.at[0, pl.ds(0, b)], sbuf[par],
                                  si[par]).wait()
            for j in range(sc_k):
                pltpu.make_async_copy(ei.at[1, pl.ds(0, qb)],
                                      dbuf[par].at[j], si[par]).wait()

        def issue_gathers(par):
            for j in range(sc_k):
                sl = pl.ds(j * qb, qb)
                pltpu.async_copy(a_src.at[sbuf[par].at[sl]],
                                 asg[par].at[sl], sg[par])
                pltpu.async_copy(a_dst.at[dbuf[par].at[j]],
                                 adg[par].at[sl], sg[par])
                pltpu.async_copy(h.at[sbuf[par].at[sl]],
                                 hgb[par].at[sl], sg[par])

        def wait_gathers(par):
            for j in range(sc_k):
                sl = pl.ds(j * qb, qb)
                pltpu.make_async_copy(a_src.at[sbuf[par].at[sl]],
                                      asg[par].at[sl], sg[par]).wait()
                pltpu.make_async_copy(a_dst.at[dbuf[par].at[j]],
                                      adg[par].at[sl], sg[par]).wait()
                pltpu.make_async_copy(h.at[sbuf[par].at[sl]],
                                      hgb[par].at[sl], sg[par]).wait()

        def wait_scatter(par):
            for j in range(sc_k):
                pltpu.make_async_copy(ob.at[pl.ds(j * qb, qb)],
                                      acc.at[dbuf[par].at[j]], ss).wait()

        def compute_scatter(par):
            wait_scatter(par)
            sh = shb[...]
            hgp = hgb[par]

            @plsc.parallel_loop(0, b, unroll=8)
            def edge(i):
                ev = asg[par][i, :] + adg[par][i, :]
                ev = jnp.maximum(ev, 0.2 * ev)
                p = jnp.exp(ev - sh)
                ob[i, pl.ds(hc, 16)] = p
                for k in range(nh):
                    ph = lax.gather(
                        p, jnp.full((16, 1), k, jnp.int32),
                        dimension_numbers=lax.GatherDimensionNumbers(
                            offset_dims=(), collapsed_slice_dims=(0,),
                            start_index_map=(0,)),
                        slice_sizes=(1,),
                        mode=lax.GatherScatterMode.PROMISE_IN_BOUNDS)
                    ob[i, pl.ds(k * 16, 16)] = ph * hgp[i, pl.ds(k * 16, 16)]

            for j in range(sc_k):
                pltpu.async_copy(ob.at[pl.ds(j * qb, qb)],
                                 acc.at[dbuf[par].at[j]], ss, add=True)

        pltpu.sync_copy(ei.at[0, pl.ds(ebase, b)], sbuf[0])
        for j in range(sc_k):
            pltpu.sync_copy(ei.at[1, pl.ds(ebase + j * qb, qb)],
                            dbuf[0].at[j])
        issue_gathers(0)
        issue_idx(1, 1)
        for j in range(sc_k):
            pltpu.async_copy(ob.at[pl.ds(j * qb, qb)],
                             acc.at[dbuf[0].at[j]], ss, add=True)

        def sub_body(r, par):
            wait_idx(1 - par)
            issue_gathers(1 - par)
            wait_gathers(par)
            compute_scatter(par)
            issue_idx(lax.rem(r + 2, rnds), par)

        def pair_body(rr, _):
            sub_body(2 * rr, 0)
            sub_body(2 * rr + 1, 1)
            return 0

        lax.fori_loop(0, (rnds - 1) // 2, pair_body, 0)
        wait_gathers(0)
        compute_scatter(0)
        wait_idx(1)
        wait_scatter(0)
        plsc.subcore_barrier()

        for z in range(rpt // qb):
            r0 = row0 + z * qb
            pltpu.sync_copy(acc.at[pl.ds(r0, qb)], part.at[c, pl.ds(r0, qb)])

    return edge_pass


def kernel(x, edge_index, W1, att_src1, att_dst1, b1,
           W2, att_src2, att_dst2, b2):
    n, _ = x.shape
    e = edge_index.shape[1]
    heads, ch = att_src1.shape
    hc1 = heads * ch
    grid = (n // _R,)

    ar = jnp.arange(hc1)
    ssrc1 = jnp.zeros((hc1, 16), jnp.float32).at[ar, ar // ch].set(
        att_src1.reshape(-1))
    sdst1 = jnp.zeros((hc1, 16), jnp.float32).at[ar, ar // ch].set(
        att_dst1.reshape(-1))
    r8 = jnp.zeros((heads, hc1), jnp.float32).at[ar // ch, ar].set(1.0)
    ms2 = jnp.zeros((ch, 16), jnp.float32).at[jnp.arange(ch), 0].set(
        att_src2[0])
    md2 = jnp.zeros((ch, 16), jnp.float32).at[jnp.arange(ch), 0].set(
        att_dst2[0])

    f32 = jnp.float32
    h1, as1, ad1, shift1 = pl.pallas_call(
        _prep1_body,
        grid=grid,
        in_specs=[
            pl.BlockSpec((_R, 128), lambda i: (i, 0)),
            pl.BlockSpec((128, 128), lambda i: (0, 0)),
            pl.BlockSpec((128, 16), lambda i: (0, 0)),
            pl.BlockSpec((128, 16), lambda i: (0, 0)),
        ],
        out_specs=[
            pl.BlockSpec((_R, 128), lambda i: (i, 0)),
            pl.BlockSpec((_R, 16), lambda i: (i, 0)),
            pl.BlockSpec((_R, 16), lambda i: (i, 0)),
            pl.BlockSpec((1, 16), lambda i: (0, 0)),
        ],
        out_shape=[
            jax.ShapeDtypeStruct((n, 128), f32),
            jax.ShapeDtypeStruct((n, 16), f32),
            jax.ShapeDtypeStruct((n, 16), f32),
            jax.ShapeDtypeStruct((1, 16), f32),
        ],
        scratch_shapes=[pltpu.VMEM((1, 16), f32), pltpu.VMEM((1, 16), f32)],
    )(x, W1, ssrc1, sdst1)

    part1 = _make_edge_pass(n, e, hc1, 1)(
        edge_index, as1, ad1, h1, shift1.reshape(-1))

    h2, as2, ad2, shift2 = pl.pallas_call(
        _prep2_body,
        grid=grid,
        in_specs=[
            pl.BlockSpec((2, _R, 144), lambda i: (0, i, 0)),
            pl.BlockSpec((1, 128), lambda i: (0, 0)),
            pl.BlockSpec((128, 16), lambda i: (0, 0)),
            pl.BlockSpec((8, 128), lambda i: (0, 0)),
            pl.BlockSpec((16, 16), lambda i: (0, 0)),
            pl.BlockSpec((16, 16), lambda i: (0, 0)),
        ],
        out_specs=[
            pl.BlockSpec((_R, 16), lambda i: (i, 0)),
            pl.BlockSpec((_R, 16), lambda i: (i, 0)),
            pl.BlockSpec((_R, 16), lambda i: (i, 0)),
            pl.BlockSpec((1, 16), lambda i: (0, 0)),
        ],
        out_shape=[
            jax.ShapeDtypeStruct((n, 16), f32),
            jax.ShapeDtypeStruct((n, 16), f32),
            jax.ShapeDtypeStruct((n, 16), f32),
            jax.ShapeDtypeStruct((1, 16), f32),
        ],
        scratch_shapes=[pltpu.VMEM((1, 16), f32), pltpu.VMEM((1, 16), f32)],
    )(part1, b1.reshape(1, -1), W2, r8, ms2, md2)

    part2 = _make_edge_pass(n, e, ch, 5)(
        edge_index, as2, ad2, h2, shift2.reshape(-1))

    out = pl.pallas_call(
        _final_body,
        grid=grid,
        in_specs=[
            pl.BlockSpec((2, _R, 32), lambda i: (0, i, 0)),
            pl.BlockSpec((1, 16), lambda i: (0, 0)),
        ],
        out_specs=pl.BlockSpec((_R, 16), lambda i: (i, 0)),
        out_shape=jax.ShapeDtypeStruct((n, 16), f32),
    )(part2, b2.reshape(1, -1))
    return out
